# baseline (speedup 1.0000x reference)
import functools

import jax
import jax.numpy as jnp
from jax import lax
from jax.experimental import pallas as pl
from jax.experimental.pallas import tpu as pltpu
from jax.experimental.pallas import tpu_sc as plsc

N = 10000
E = 320000
D = 128
FS = 64
OUT = 10
G = 64

NC = 2
NS = 16
NW = NC * NS
K = 320

NBUF = 1
EPT = ((E + NW * K * NBUF - 1) // (NW * K * NBUF)) * K * NBUF
E_PAD = EPT * NW
CHUNKS = EPT // K
GROUPS = CHUNKS // 2
NPAD = ((N + 1 + NS * 8 - 1) // (NS * 8)) * NS * 8
RPT = NPAD // NS
DEGW = 8

_mesh = functools.partial(
    plsc.VectorSubcoreMesh, core_axis_name="c", subcore_axis_name="s")

_sc_params = pltpu.CompilerParams(use_tc_tiling_on_sc=False)


def _deg_body(dst_hbm, zeros_hbm, ones_hbm, out_hbm,
              ones_v, idx_v, acc, sem_i, sem_s):
    c = lax.axis_index("c")
    s = lax.axis_index("s")
    wid = c * NS + s
    row0 = s * RPT
    icp = pltpu.async_copy(dst_hbm.at[pl.ds(wid * EPT, EPT)], idx_v, sem_i)
    pltpu.sync_copy(zeros_hbm.at[pl.ds(row0, RPT)], acc.at[pl.ds(row0, RPT)])
    pltpu.sync_copy(ones_hbm, ones_v)
    icp.wait()
    plsc.subcore_barrier()

    def body(j, carry):
        pltpu.async_copy(
            ones_v, acc.at[idx_v.at[pl.ds(j * K, K)]], sem_s, add=True).wait()
        return carry

    lax.fori_loop(0, CHUNKS, body, 0)
    plsc.subcore_barrier()
    pltpu.sync_copy(acc.at[pl.ds(row0, RPT)], out_hbm.at[c, pl.ds(row0, RPT)])


_deg_call = pl.kernel(
    _deg_body,
    out_type=jax.ShapeDtypeStruct((NC, NPAD, DEGW), jnp.float32),
    mesh=_mesh(),
    compiler_params=_sc_params,
    scratch_types=[
        pltpu.VMEM((K, DEGW), jnp.float32),
        pltpu.VMEM((EPT,), jnp.int32),
        pltpu.VMEM_SHARED((NPAD, DEGW), jnp.float32),
        pltpu.SemaphoreType.DMA,
        pltpu.SemaphoreType.DMA,
    ],
)


def _agg_body(g_hbm, eidx_hbm, zeros_hbm, out_hbm,
              eidx, rows, g_s, acc,
              sem_i, sem_g0, sem_g1, sem_s0, sem_s1):
    c = lax.axis_index("c")
    s = lax.axis_index("s")
    wid = c * NS + s
    base = wid * CHUNKS * 2 * K
    row0 = s * RPT
    sem_g = (sem_g0, sem_g1)
    sem_s = (sem_s0, sem_s1)
    pltpu.sync_copy(zeros_hbm.at[pl.ds(row0, RPT)], acc.at[pl.ds(row0, RPT)])
    pltpu.sync_copy(g_hbm.at[pl.ds(row0, RPT)], g_s.at[pl.ds(row0, RPT)])
    plsc.subcore_barrier()

    def fetch_idx(j, b):
        pltpu.sync_copy(
            eidx_hbm.at[pl.ds(base + j * 2 * K, 2 * K)], eidx.at[b])

    def gather(j, b):
        return pltpu.async_copy(
            g_s.at[eidx.at[b].at[pl.ds(0, K)]], rows.at[b], sem_g[b])

    def scatter(b):
        return pltpu.async_copy(
            rows.at[b], acc.at[eidx.at[b].at[pl.ds(K, K)]], sem_s[b],
            add=True)

    fetch_idx(0, 0)
    g0 = gather(0, 0)
    fetch_idx(1, 1)
    g1 = gather(1, 1)
    g0.wait()
    scatter(0)
    g1.wait()
    scatter(1)

    def body(g, carry):
        j0 = 2 * g
        for b in range(2):
            pltpu.make_async_copy(
                rows.at[b], acc.at[eidx.at[b].at[pl.ds(K, K)]],
                sem_s[b]).wait()
            fetch_idx(j0 + b, b)
            gather(j0 + b, b)
        for b in range(2):
            pltpu.make_async_copy(
                g_s.at[eidx.at[b].at[pl.ds(0, K)]], rows.at[b],
                sem_g[b]).wait()
            scatter(b)
        return carry

    lax.fori_loop(1, GROUPS, body, 0)
    for b in range(2):
        pltpu.make_async_copy(
            rows.at[b], acc.at[eidx.at[b].at[pl.ds(K, K)]],
            sem_s[b]).wait()
    plsc.subcore_barrier()
    pltpu.sync_copy(acc.at[pl.ds(row0, RPT)], out_hbm.at[c, pl.ds(row0, RPT)])


_agg_call = pl.kernel(
    _agg_body,
    out_type=jax.ShapeDtypeStruct((NC, NPAD, FS), jnp.float32),
    mesh=_mesh(),
    compiler_params=_sc_params,
    scratch_types=[
        pltpu.VMEM((2, 2 * K), jnp.int32),
        pltpu.VMEM((2, K, FS), jnp.float32),
        pltpu.VMEM_SHARED((NPAD, FS), jnp.float32),
        pltpu.VMEM_SHARED((NPAD, FS), jnp.float32),
        pltpu.SemaphoreType.DMA,
        pltpu.SemaphoreType.DMA,
        pltpu.SemaphoreType.DMA,
        pltpu.SemaphoreType.DMA,
        pltpu.SemaphoreType.DMA,
    ],
)


def _dinv_from(degp_ref):
    deg = degp_ref[0, :, 0] + degp_ref[1, :, 0]
    return jnp.where(deg > 0.0, lax.rsqrt(jnp.maximum(deg, 1e-30)), 0.0)


def _tc_matmul1(x_ref, w_ref, h_ref):
    h_ref[...] = jnp.dot(
        x_ref[...], w_ref[...], preferred_element_type=jnp.float32)


def _tc_scale1(h_ref, degp_ref, g_ref):
    dinv = _dinv_from(degp_ref)
    g_ref[...] = h_ref[...] * dinv[:, None]


def _tc_layer(aggp_ref, degp_ref, b_ref, w_ref, x_ref, g_ref):
    dinv = _dinv_from(degp_ref)
    agg = aggp_ref[0] + aggp_ref[1]
    xk = jnp.maximum(agg * dinv[:, None] + b_ref[...][None, :], 0.0)
    x_ref[...] = xk
    g_ref[...] = jnp.dot(
        xk, w_ref[...], preferred_element_type=jnp.float32) * dinv[:, None]


def _tc_final(aggp_ref, degp_ref, b_ref, x1_ref, x2_ref, batch_ref,
              wf_ref, bf_ref, out_ref):
    dinv = _dinv_from(degp_ref)
    x3 = jnp.maximum(
        (aggp_ref[0] + aggp_ref[1]) * dinv[:, None] + b_ref[...][None, :], 0.0)
    xc = jnp.concatenate([x1_ref[...], x2_ref[...], x3], axis=1)[:N]
    seg = lax.broadcasted_iota(jnp.int32, (G, N), 0)
    p = (batch_ref[...][None, :] == seg).astype(jnp.float32)
    sums = jnp.dot(p, xc, preferred_element_type=jnp.float32)
    cnt = jnp.sum(p, axis=1)
    pooled = sums / jnp.maximum(cnt, 1.0)[:, None]
    logits = jnp.dot(
        pooled, wf_ref[...], preferred_element_type=jnp.float32
    ) + bf_ref[...][None, :]
    m = jnp.max(logits, axis=1, keepdims=True)
    ex = jnp.exp(logits - m)
    out_ref[...] = ex / jnp.sum(ex, axis=1, keepdims=True)


def kernel(x, edge_index, batch, W1, b1, W2, b2, W3, b3, Wf, bf):
    f32 = jnp.float32
    src = edge_index[0]
    dst = edge_index[1]
    pad_idx = jnp.full((E_PAD - E,), N, jnp.int32)
    src_p = jnp.concatenate([src, pad_idx])
    dst_p = jnp.concatenate([dst, pad_idx])
    eidx_p = jnp.stack(
        [src_p.reshape(NW, CHUNKS, K), dst_p.reshape(NW, CHUNKS, K)],
        axis=2).reshape(-1)
    x_pad = jnp.zeros((NPAD, D), f32).at[:N].set(x)
    zeros_deg = jnp.zeros((NPAD, DEGW), f32)
    ones_deg = jnp.ones((K, DEGW), f32)
    zeros_fs = jnp.zeros((NPAD, FS), f32)

    h1 = pl.pallas_call(
        _tc_matmul1,
        out_shape=jax.ShapeDtypeStruct((NPAD, FS), f32),
    )(x_pad, W1)

    degp = _deg_call(dst_p, zeros_deg, ones_deg)

    g1 = pl.pallas_call(
        _tc_scale1,
        out_shape=jax.ShapeDtypeStruct((NPAD, FS), f32),
    )(h1, degp)

    aggp1 = _agg_call(g1, eidx_p, zeros_fs)

    x1, g2 = pl.pallas_call(
        _tc_layer,
        out_shape=(jax.ShapeDtypeStruct((NPAD, FS), f32),
                   jax.ShapeDtypeStruct((NPAD, FS), f32)),
    )(aggp1, degp, b1, W2)

    aggp2 = _agg_call(g2, eidx_p, zeros_fs)

    x2, g3 = pl.pallas_call(
        _tc_layer,
        out_shape=(jax.ShapeDtypeStruct((NPAD, FS), f32),
                   jax.ShapeDtypeStruct((NPAD, FS), f32)),
    )(aggp2, degp, b2, W3)

    aggp3 = _agg_call(g3, eidx_p, zeros_fs)

    return pl.pallas_call(
        _tc_final,
        out_shape=jax.ShapeDtypeStruct((G, OUT), f32),
    )(aggp3, degp, b3, x1, x2, batch, Wf, bf)


# speedup vs baseline: 1.0109x; 1.0109x over previous
import functools

import jax
import jax.numpy as jnp
from jax import lax
from jax.experimental import pallas as pl
from jax.experimental.pallas import tpu as pltpu
from jax.experimental.pallas import tpu_sc as plsc

N = 10000
E = 320000
D = 128
FS = 64
OUT = 10
G = 64

NC = 2
NS = 16
NW = NC * NS
K = 256

NBUF = 1
EPT = ((E + NW * K * NBUF - 1) // (NW * K * NBUF)) * K * NBUF
E_PAD = EPT * NW
CHUNKS = EPT // K
GROUPS = CHUNKS // 2
NPAD = ((N + 1 + NS * 8 - 1) // (NS * 8)) * NS * 8
RPT = NPAD // NS
DEGW = 8

_mesh = functools.partial(
    plsc.VectorSubcoreMesh, core_axis_name="c", subcore_axis_name="s")

_sc_params = pltpu.CompilerParams(use_tc_tiling_on_sc=False)


def _deg_body(dst_hbm, zeros_hbm, ones_hbm, out_hbm,
              ones_v, idx_v, acc, sem_i, sem_s):
    c = lax.axis_index("c")
    s = lax.axis_index("s")
    wid = c * NS + s
    row0 = s * RPT
    icp = pltpu.async_copy(dst_hbm.at[pl.ds(wid * EPT, EPT)], idx_v, sem_i)
    pltpu.sync_copy(zeros_hbm.at[pl.ds(row0, RPT)], acc.at[pl.ds(row0, RPT)])
    pltpu.sync_copy(ones_hbm, ones_v)
    icp.wait()
    plsc.subcore_barrier()

    def body(j, carry):
        pltpu.async_copy(
            ones_v, acc.at[idx_v.at[pl.ds(j * K, K)]], sem_s, add=True).wait()
        return carry

    lax.fori_loop(0, CHUNKS, body, 0)
    plsc.subcore_barrier()
    pltpu.sync_copy(acc.at[pl.ds(row0, RPT)], out_hbm.at[c, pl.ds(row0, RPT)])


_deg_call = pl.kernel(
    _deg_body,
    out_type=jax.ShapeDtypeStruct((NC, NPAD, DEGW), jnp.float32),
    mesh=_mesh(),
    compiler_params=_sc_params,
    scratch_types=[
        pltpu.VMEM((K, DEGW), jnp.float32),
        pltpu.VMEM((EPT,), jnp.int32),
        pltpu.VMEM_SHARED((NPAD, DEGW), jnp.float32),
        pltpu.SemaphoreType.DMA,
        pltpu.SemaphoreType.DMA,
    ],
)


def _agg_body(g_hbm, eidx_hbm, zeros_hbm, out_hbm,
              eidx, rows, g_s, acc,
              sem_i, sem_g0, sem_g1, sem_s0, sem_s1):
    c = lax.axis_index("c")
    s = lax.axis_index("s")
    wid = c * NS + s
    base = wid * CHUNKS * 2 * K
    row0 = s * RPT
    sem_g = (sem_g0, sem_g1)
    sem_s = (sem_s0, sem_s1)
    pltpu.sync_copy(zeros_hbm.at[pl.ds(row0, RPT)], acc.at[pl.ds(row0, RPT)])
    pltpu.sync_copy(g_hbm.at[pl.ds(row0, RPT)], g_s.at[pl.ds(row0, RPT)])
    plsc.subcore_barrier()

    def fetch_idx(j, b):
        pltpu.sync_copy(
            eidx_hbm.at[pl.ds(base + j * 2 * K, 2 * K)], eidx.at[b])

    def gather(j, b):
        return pltpu.async_copy(
            g_s.at[eidx.at[b].at[pl.ds(0, K)]], rows.at[b], sem_g[b])

    def scatter(b):
        return pltpu.async_copy(
            rows.at[b], acc.at[eidx.at[b].at[pl.ds(K, K)]], sem_s[b],
            add=True)

    fetch_idx(0, 0)
    g0 = gather(0, 0)
    fetch_idx(1, 1)
    g1 = gather(1, 1)
    g0.wait()
    scatter(0)
    g1.wait()
    scatter(1)

    def body(g, carry):
        j0 = 2 * g
        for b in range(2):
            pltpu.make_async_copy(
                rows.at[b], acc.at[eidx.at[b].at[pl.ds(K, K)]],
                sem_s[b]).wait()
            fetch_idx(j0 + b, b)
            gather(j0 + b, b)
        for b in range(2):
            pltpu.make_async_copy(
                g_s.at[eidx.at[b].at[pl.ds(0, K)]], rows.at[b],
                sem_g[b]).wait()
            scatter(b)
        return carry

    lax.fori_loop(1, GROUPS, body, 0)
    for b in range(2):
        pltpu.make_async_copy(
            rows.at[b], acc.at[eidx.at[b].at[pl.ds(K, K)]],
            sem_s[b]).wait()
    plsc.subcore_barrier()
    pltpu.sync_copy(acc.at[pl.ds(row0, RPT)], out_hbm.at[c, pl.ds(row0, RPT)])


_agg_call = pl.kernel(
    _agg_body,
    out_type=jax.ShapeDtypeStruct((NC, NPAD, FS), jnp.float32),
    mesh=_mesh(),
    compiler_params=_sc_params,
    scratch_types=[
        pltpu.VMEM((2, 2 * K), jnp.int32),
        pltpu.VMEM((2, K, FS), jnp.float32),
        pltpu.VMEM_SHARED((NPAD, FS), jnp.float32),
        pltpu.VMEM_SHARED((NPAD, FS), jnp.float32),
        pltpu.SemaphoreType.DMA,
        pltpu.SemaphoreType.DMA,
        pltpu.SemaphoreType.DMA,
        pltpu.SemaphoreType.DMA,
        pltpu.SemaphoreType.DMA,
    ],
)


def _dinv_from(degp_ref):
    deg = degp_ref[0, :, 0] + degp_ref[1, :, 0]
    return jnp.where(deg > 0.0, lax.rsqrt(jnp.maximum(deg, 1e-30)), 0.0)


def _tc_matmul1(x_ref, w_ref, h_ref):
    h_ref[...] = jnp.dot(
        x_ref[...], w_ref[...], preferred_element_type=jnp.float32)


def _tc_scale1(h_ref, degp_ref, g_ref):
    dinv = _dinv_from(degp_ref)
    g_ref[...] = h_ref[...] * dinv[:, None]


def _tc_layer(aggp_ref, degp_ref, b_ref, w_ref, x_ref, g_ref):
    dinv = _dinv_from(degp_ref)
    agg = aggp_ref[0] + aggp_ref[1]
    xk = jnp.maximum(agg * dinv[:, None] + b_ref[...][None, :], 0.0)
    x_ref[...] = xk
    g_ref[...] = jnp.dot(
        xk, w_ref[...], preferred_element_type=jnp.float32) * dinv[:, None]


def _tc_final(aggp_ref, degp_ref, b_ref, x1_ref, x2_ref, batch_ref,
              wf_ref, bf_ref, out_ref):
    dinv = _dinv_from(degp_ref)
    x3 = jnp.maximum(
        (aggp_ref[0] + aggp_ref[1]) * dinv[:, None] + b_ref[...][None, :], 0.0)
    xc = jnp.concatenate([x1_ref[...], x2_ref[...], x3], axis=1)[:N]
    seg = lax.broadcasted_iota(jnp.int32, (G, N), 0)
    p = (batch_ref[...][None, :] == seg).astype(jnp.float32)
    sums = jnp.dot(p, xc, preferred_element_type=jnp.float32)
    cnt = jnp.sum(p, axis=1)
    pooled = sums / jnp.maximum(cnt, 1.0)[:, None]
    logits = jnp.dot(
        pooled, wf_ref[...], preferred_element_type=jnp.float32
    ) + bf_ref[...][None, :]
    m = jnp.max(logits, axis=1, keepdims=True)
    ex = jnp.exp(logits - m)
    out_ref[...] = ex / jnp.sum(ex, axis=1, keepdims=True)


def kernel(x, edge_index, batch, W1, b1, W2, b2, W3, b3, Wf, bf):
    f32 = jnp.float32
    src = edge_index[0]
    dst = edge_index[1]
    pad_idx = jnp.full((E_PAD - E,), N, jnp.int32)
    src_p = jnp.concatenate([src, pad_idx])
    dst_p = jnp.concatenate([dst, pad_idx])
    eidx_p = jnp.stack(
        [src_p.reshape(NW, CHUNKS, K), dst_p.reshape(NW, CHUNKS, K)],
        axis=2).reshape(-1)
    x_pad = jnp.zeros((NPAD, D), f32).at[:N].set(x)
    zeros_deg = jnp.zeros((NPAD, DEGW), f32)
    ones_deg = jnp.ones((K, DEGW), f32)
    zeros_fs = jnp.zeros((NPAD, FS), f32)

    h1 = pl.pallas_call(
        _tc_matmul1,
        out_shape=jax.ShapeDtypeStruct((NPAD, FS), f32),
    )(x_pad, W1)

    degp = _deg_call(dst_p, zeros_deg, ones_deg)

    g1 = pl.pallas_call(
        _tc_scale1,
        out_shape=jax.ShapeDtypeStruct((NPAD, FS), f32),
    )(h1, degp)

    aggp1 = _agg_call(g1, eidx_p, zeros_fs)

    x1, g2 = pl.pallas_call(
        _tc_layer,
        out_shape=(jax.ShapeDtypeStruct((NPAD, FS), f32),
                   jax.ShapeDtypeStruct((NPAD, FS), f32)),
    )(aggp1, degp, b1, W2)

    aggp2 = _agg_call(g2, eidx_p, zeros_fs)

    x2, g3 = pl.pallas_call(
        _tc_layer,
        out_shape=(jax.ShapeDtypeStruct((NPAD, FS), f32),
                   jax.ShapeDtypeStruct((NPAD, FS), f32)),
    )(aggp2, degp, b2, W3)

    aggp3 = _agg_call(g3, eidx_p, zeros_fs)

    return pl.pallas_call(
        _tc_final,
        out_shape=jax.ShapeDtypeStruct((G, OUT), f32),
    )(aggp3, degp, b3, x1, x2, batch, Wf, bf)
